# baseline (speedup 1.0000x reference)
import functools

import jax
import jax.numpy as jnp
from jax import lax
from jax.experimental import pallas as pl
from jax.experimental.pallas import tpu as pltpu
from jax.experimental.pallas import tpu_sc as plsc

_INPUT_DIM = 1000000
_OUTPUT_DIM = 32
_BATCH = 16384
_N_FIELDS = 26

_NB = _BATCH * _N_FIELDS
_NC, _NS = 2, 16
_NW = _NC * _NS
_BPW = _NB // _NW
_BLK = 128
_BLKS_PER_W = _BATCH // _BLK // _NW
_UNITS = _N_FIELDS * _BLKS_PER_W


@functools.partial(
    pl.kernel,
    out_type=jax.ShapeDtypeStruct(
        (_N_FIELDS, _OUTPUT_DIM // 8, _BATCH // _BLK, 8, _BLK), jnp.float32),
    mesh=plsc.VectorSubcoreMesh(core_axis_name="c", subcore_axis_name="s"),
    compiler_params=pltpu.CompilerParams(use_tc_tiling_on_sc=False,
                                         needs_layout_passes=False),
    scratch_types=[
        pltpu.VMEM((_BPW,), jnp.int32),
        pltpu.VMEM((_BLK,), jnp.int32),
        pltpu.VMEM((_BLK,), jnp.int32),
        pltpu.VMEM((_BLK, _OUTPUT_DIM), jnp.float32),
        pltpu.VMEM((_BLK, _OUTPUT_DIM), jnp.float32),
        pltpu.VMEM((_OUTPUT_DIM // 8, 8, _BLK), jnp.float32),
        pltpu.VMEM((_OUTPUT_DIM // 8, 8, _BLK), jnp.float32),
        pltpu.SemaphoreType.DMA,
        pltpu.SemaphoreType.DMA,
        pltpu.SemaphoreType.DMA,
        pltpu.SemaphoreType.DMA,
    ],
)
def _emb_lookup(table_hbm, idx_hbm, out_hbm, idx_v, icol0, icol1,
                rows0, rows1, trans0, trans1, g0, g1, w0, w1):
    wid = lax.axis_index("s") * _NC + lax.axis_index("c")
    pltpu.sync_copy(idx_hbm.at[pl.ds(wid * _BPW, _BPW)], idx_v)

    icol = (icol0, icol1)
    rows = (rows0, rows1)
    trans = (trans0, trans1)
    gsem = (g0, g1)
    wsem = (w0, w1)

    def unit_ids(u):
        if isinstance(u, int):
            return u >> 2, u & 3
        return (lax.shift_right_logical(u, 2), lax.bitwise_and(u, 3))

    def build_icol(u, p):
        b2, blk = unit_ids(u)
        base = blk * (_BLK * _N_FIELDS) + b2
        step = lax.iota(jnp.int32, 16) * _N_FIELDS
        for g in range(8):
            vals = plsc.load_gather(idx_v, [step + (base + g * 16 * _N_FIELDS)])
            icol[p][pl.ds(g * 16, 16)] = vals

    def fire_gather(p):
        pltpu.make_async_copy(table_hbm.at[icol[p]], rows[p], gsem[p]).start()

    def drain_gather(p):
        pltpu.make_async_copy(table_hbm.at[pl.ds(0, _BLK)], rows[p],
                              gsem[p]).wait()

    def transpose(p):
        def tbody(g, carry):
            row_ids = lax.iota(jnp.int32, 16) + g * 16
            for d in range(_OUTPUT_DIM):
                vals = plsc.load_gather(
                    rows[p], [row_ids, jnp.full((16,), d, jnp.int32)])
                trans[p][d // 8, d % 8, pl.ds(g * 16, 16)] = vals
            return carry
        lax.fori_loop(0, 8, tbody, 0)

    def fire_write(u, p):
        b2, blk = unit_ids(u)
        blkg = wid * _BLKS_PER_W + blk
        pltpu.make_async_copy(trans[p], out_hbm.at[b2, :, blkg],
                              wsem[p]).start()

    def drain_write(p):
        pltpu.make_async_copy(trans[p], out_hbm.at[0, :, 0], wsem[p]).wait()

    for p in (0, 1):
        build_icol(p, p)
        fire_gather(p)
    for p in (0, 1):
        drain_gather(p)
        transpose(p)
        fire_write(p, p)
        build_icol(p + 2, p)
        fire_gather(p)

    def body(i, carry):
        for p in (0, 1):
            u = 2 * i + p
            drain_gather(p)
            drain_write(p)
            transpose(p)
            fire_write(u, p)
            build_icol(u + 2, p)
            fire_gather(p)
        return carry

    lax.fori_loop(1, (_UNITS - 2) // 2, body, 0)

    for p in (0, 1):
        drain_gather(p)
        drain_write(p)
        transpose(p)
        fire_write(_UNITS - 2 + p, p)
    for p in (0, 1):
        drain_write(p)


def kernel(inputs, embeddings):
    idx = inputs.reshape(-1).astype(jnp.int32)
    out5 = _emb_lookup(embeddings, idx)
    out = out5.transpose(2, 4, 0, 1, 3).reshape(_BATCH, _N_FIELDS, _OUTPUT_DIM)
    return out


# speedup vs baseline: 1.3349x; 1.3349x over previous
import functools

import jax
import jax.numpy as jnp
from jax import lax
from jax.experimental import pallas as pl
from jax.experimental.pallas import tpu as pltpu
from jax.experimental.pallas import tpu_sc as plsc

_INPUT_DIM = 1000000
_OUTPUT_DIM = 32
_BATCH = 16384
_N_FIELDS = 26

_NB = _BATCH * _N_FIELDS
_NC, _NS = 2, 16
_NW = _NC * _NS
_BPW = _NB // _NW
_BLK = 128
_BLKS_PER_W = _BATCH // _BLK // _NW
_UNITS = _N_FIELDS * _BLKS_PER_W


@functools.partial(
    pl.kernel,
    out_type=jax.ShapeDtypeStruct(
        (_N_FIELDS, _OUTPUT_DIM // 8, _BATCH // _BLK, 8, _BLK), jnp.float32),
    mesh=plsc.VectorSubcoreMesh(core_axis_name="c", subcore_axis_name="s"),
    compiler_params=pltpu.CompilerParams(use_tc_tiling_on_sc=False,
                                         needs_layout_passes=False),
    scratch_types=[
        pltpu.VMEM((_BPW,), jnp.int32),
        pltpu.VMEM((_BLK,), jnp.int32),
        pltpu.VMEM((_BLK,), jnp.int32),
        pltpu.VMEM((_BLK, _OUTPUT_DIM), jnp.float32),
        pltpu.VMEM((_BLK, _OUTPUT_DIM), jnp.float32),
        pltpu.VMEM((_OUTPUT_DIM // 8, 8, _BLK), jnp.float32),
        pltpu.VMEM((_OUTPUT_DIM // 8, 8, _BLK), jnp.float32),
        pltpu.SemaphoreType.DMA,
        pltpu.SemaphoreType.DMA,
        pltpu.SemaphoreType.DMA,
        pltpu.SemaphoreType.DMA,
    ],
)
def _emb_lookup(table_hbm, idx_hbm, out_hbm, idx_v, icol0, icol1,
                rows0, rows1, trans0, trans1, g0, g1, w0, w1):
    wid = lax.axis_index("s") * _NC + lax.axis_index("c")
    pltpu.sync_copy(idx_hbm.at[pl.ds(wid * _BPW, _BPW)], idx_v)

    icol = (icol0, icol1)
    rows = (rows0, rows1)
    trans = (trans0, trans1)
    gsem = (g0, g1)
    wsem = (w0, w1)

    def unit_ids(u):
        if isinstance(u, int):
            return u >> 2, u & 3
        return (lax.shift_right_logical(u, 2), lax.bitwise_and(u, 3))

    def build_icol(u, p):
        b2, blk = unit_ids(u)
        base = blk * (_BLK * _N_FIELDS) + b2
        step = lax.iota(jnp.int32, 16) * _N_FIELDS
        for g in range(8):
            vals = plsc.load_gather(idx_v, [step + (base + g * 16 * _N_FIELDS)])
            icol[p][pl.ds(g * 16, 16)] = vals

    def fire_gather(p):
        pltpu.make_async_copy(table_hbm.at[icol[p]], rows[p], gsem[p]).start()

    def drain_gather(p):
        pltpu.make_async_copy(table_hbm.at[pl.ds(0, _BLK)], rows[p],
                              gsem[p]).wait()

    def transpose(p):
        iota = lax.iota(jnp.int32, 16)

        def tbody(g, carry):
            row_ids = iota + g * 16
            for d0 in range(_OUTPUT_DIM):
                d_vec = lax.bitwise_and(iota + d0, _OUTPUT_DIM - 1)
                vals = plsc.load_gather(rows[p], [row_ids, d_vec])
                r_vec = lax.shift_right_logical(d_vec, 3)
                s_vec = lax.bitwise_and(d_vec, 7)
                plsc.store_scatter(trans[p], [r_vec, s_vec, row_ids], vals)
            return carry

        lax.fori_loop(0, 8, tbody, 0)

    def fire_write(u, p):
        b2, blk = unit_ids(u)
        blkg = wid * _BLKS_PER_W + blk
        pltpu.make_async_copy(trans[p], out_hbm.at[b2, :, blkg],
                              wsem[p]).start()

    def drain_write(p):
        pltpu.make_async_copy(trans[p], out_hbm.at[0, :, 0], wsem[p]).wait()

    for p in (0, 1):
        build_icol(p, p)
        fire_gather(p)
    for p in (0, 1):
        drain_gather(p)
        transpose(p)
        fire_write(p, p)
        build_icol(p + 2, p)
        fire_gather(p)

    def body(i, carry):
        for p in (0, 1):
            u = 2 * i + p
            drain_gather(p)
            drain_write(p)
            transpose(p)
            fire_write(u, p)
            build_icol(u + 2, p)
            fire_gather(p)
        return carry

    lax.fori_loop(1, (_UNITS - 2) // 2, body, 0)

    for p in (0, 1):
        drain_gather(p)
        drain_write(p)
        transpose(p)
        fire_write(_UNITS - 2 + p, p)
    for p in (0, 1):
        drain_write(p)


def kernel(inputs, embeddings):
    idx = inputs.reshape(-1).astype(jnp.int32)
    out5 = _emb_lookup(embeddings, idx)
    out = out5.transpose(2, 4, 0, 1, 3).reshape(_BATCH, _N_FIELDS, _OUTPUT_DIM)
    return out


# speedup vs baseline: 1.3349x; 1.0000x over previous
import functools

import jax
import jax.numpy as jnp
from jax import lax
from jax.experimental import pallas as pl
from jax.experimental.pallas import tpu as pltpu
from jax.experimental.pallas import tpu_sc as plsc

_INPUT_DIM = 1000000
_OUTPUT_DIM = 32
_BATCH = 16384
_N_FIELDS = 26

_NB = _BATCH * _N_FIELDS
_NC, _NS = 2, 16
_NW = _NC * _NS
_BPW = _NB // _NW
_BLK = 128
_BLKS_PER_W = _BATCH // _BLK // _NW
_UNITS = _N_FIELDS * _BLKS_PER_W


@functools.partial(
    pl.kernel,
    out_type=jax.ShapeDtypeStruct(
        (_N_FIELDS, _OUTPUT_DIM // 8, _BATCH // _BLK, 8, _BLK), jnp.float32),
    mesh=plsc.VectorSubcoreMesh(core_axis_name="c", subcore_axis_name="s"),
    compiler_params=pltpu.CompilerParams(use_tc_tiling_on_sc=False,
                                         needs_layout_passes=False),
    scratch_types=[
        pltpu.VMEM((_BPW,), jnp.int32),
        pltpu.VMEM((_BLK,), jnp.int32),
        pltpu.VMEM((_BLK,), jnp.int32),
        pltpu.VMEM((_BLK, _OUTPUT_DIM), jnp.float32),
        pltpu.VMEM((_BLK, _OUTPUT_DIM), jnp.float32),
        pltpu.VMEM((_OUTPUT_DIM // 8, 8, _BLK), jnp.float32),
        pltpu.VMEM((_OUTPUT_DIM // 8, 8, _BLK), jnp.float32),
        pltpu.SemaphoreType.DMA,
        pltpu.SemaphoreType.DMA,
        pltpu.SemaphoreType.DMA,
        pltpu.SemaphoreType.DMA,
    ],
)
def _emb_lookup(table_hbm, idx_hbm, out_hbm, idx_v, icol0, icol1,
                rows0, rows1, trans0, trans1, g0, g1, w0, w1):
    wid = lax.axis_index("s") * _NC + lax.axis_index("c")
    pltpu.sync_copy(idx_hbm.at[pl.ds(wid * _BPW, _BPW)], idx_v)

    icol = (icol0, icol1)
    rows = (rows0, rows1)
    trans = (trans0, trans1)
    gsem = (g0, g1)
    wsem = (w0, w1)

    def unit_ids(u):
        if isinstance(u, int):
            return u >> 2, u & 3
        return (lax.shift_right_logical(u, 2), lax.bitwise_and(u, 3))

    def build_icol(u, p):
        b2, blk = unit_ids(u)
        base = blk * (_BLK * _N_FIELDS) + b2
        step = lax.iota(jnp.int32, 16) * _N_FIELDS
        for g in range(8):
            vals = plsc.load_gather(idx_v, [step + (base + g * 16 * _N_FIELDS)])
            icol[p][pl.ds(g * 16, 16)] = vals

    def fire_gather(p):
        pltpu.make_async_copy(table_hbm.at[icol[p]], rows[p],
                              gsem[p]).start()

    def drain_gather(p):
        pltpu.make_async_copy(table_hbm.at[pl.ds(0, _BLK)], rows[p],
                              gsem[p]).wait()

    iota = lax.iota(jnp.int32, 16)
    d_vecs = [(iota + d0) & (_OUTPUT_DIM - 1) for d0 in range(_OUTPUT_DIM)]
    r_vecs = [d >> 3 for d in d_vecs]
    s_vecs = [d & 7 for d in d_vecs]

    def transpose(p):
        def tbody(g, carry):
            row_ids = iota + g * 16
            for d0 in range(_OUTPUT_DIM):
                vals = plsc.load_gather(rows[p], [row_ids, d_vecs[d0]])
                plsc.store_scatter(trans[p],
                                   [r_vecs[d0], s_vecs[d0], row_ids], vals)
            return carry

        lax.fori_loop(0, 8, tbody, 0)

    def fire_write(u, p):
        b2, blk = unit_ids(u)
        blkg = wid * _BLKS_PER_W + blk
        pltpu.make_async_copy(trans[p], out_hbm.at[b2, :, blkg],
                              wsem[p]).start()

    def drain_write(p):
        pltpu.make_async_copy(trans[p], out_hbm.at[0, :, 0], wsem[p]).wait()

    for p in (0, 1):
        build_icol(p, p)
        fire_gather(p)
    for p in (0, 1):
        drain_gather(p)
        transpose(p)
        fire_write(p, p)
        build_icol(p + 2, p)
        fire_gather(p)

    def body(i, carry):
        for p in (0, 1):
            u = 2 * i + p
            drain_gather(p)
            drain_write(p)
            transpose(p)
            fire_write(u, p)
            build_icol(u + 2, p)
            fire_gather(p)
        return carry

    lax.fori_loop(1, (_UNITS - 2) // 2, body, 0)

    for p in (0, 1):
        drain_gather(p)
        drain_write(p)
        transpose(p)
        fire_write(_UNITS - 2 + p, p)
    for p in (0, 1):
        drain_write(p)


def kernel(inputs, embeddings):
    idx = inputs.reshape(-1).astype(jnp.int32)
    out5 = _emb_lookup(embeddings, idx)
    out = out5.transpose(2, 4, 0, 1, 3).reshape(_BATCH, _N_FIELDS, _OUTPUT_DIM)
    return out
